# Initial kernel scaffold; baseline (speedup 1.0000x reference)
import jax
import jax.numpy as jnp
from jax.experimental import pallas as pl


def kernel(h_sent, h_type, edge_index, attn_W):
    raise NotImplementedError("write your pallas kernel here")


# speedup vs baseline: 10.2951x; 10.2951x over previous
import jax
import jax.numpy as jnp
from jax import lax
from jax.experimental import pallas as pl
from jax.experimental.pallas import tpu as pltpu
from jax.experimental.pallas import tpu_sc as plsc

N = 10000
E = 160000
D = 256
H = 128
L = 16
NS = 16
NPH = 10112
RT = NPH // NS
EROWS = 1280
RPT = EROWS // NS
K8 = H // L


def _scores_body(hs_ref, ht_ref, w_ref, s1_ref, s2_ref):
    w = w_ref[0]
    s1_ref[...] = jnp.dot(hs_ref[...], w[:D], preferred_element_type=jnp.float32)
    s2_ref[...] = jnp.dot(ht_ref[...], w[D:], preferred_element_type=jnp.float32)


def _scores(h_sent, h_type, attn_W):
    return pl.pallas_call(
        _scores_body,
        out_shape=[
            jax.ShapeDtypeStruct((N,), jnp.float32),
            jax.ShapeDtypeStruct((N,), jnp.float32),
        ],
    )(h_sent, h_type, attn_W)


def _full16(v):
    return jnp.full((L,), v, dtype=jnp.int32)


def _gat_sc_body(hs2, ht2, src2, dst2, s1h, s2h, out2,
                 acc_sh, den_sh, s1_sh, s2_sh,
                 dstb, srcc, rowb, s1row, s2row, dnrow, exrow, denbuf,
                 alrow, sem):
    c = lax.axis_index("c")
    s = lax.axis_index("s")
    zero16 = jnp.zeros((L,), jnp.float32)
    coff = c * NPH
    ebase = s * RPT

    def _zrow(i, carry):
        for k in range(K8):
            rowb[0, i, pl.ds(k * L, L)] = zero16
        return carry

    lax.fori_loop(0, H, _zrow, None)
    for k in range(K8):
        denbuf[pl.ds(k * L, L)] = zero16

    for k4 in range(4):
        pltpu.sync_copy(rowb.at[0], acc_sh.at[pl.ds(s * RT + k4 * H, H)])
        pltpu.sync_copy(denbuf, den_sh.at[pl.ds(s * RT + k4 * H, H)])
    pltpu.sync_copy(rowb.at[0, pl.ds(0, RT - 4 * H)],
                    acc_sh.at[pl.ds(s * RT + 4 * H, RT - 4 * H)])
    pltpu.sync_copy(denbuf.at[pl.ds(0, RT - 4 * H)],
                    den_sh.at[pl.ds(s * RT + 4 * H, RT - 4 * H)])

    @pl.when(s < 10)
    def _stage_scores():
        for i8, sz in [(0, H), (1, H), (2, H), (3, H), (4, H), (5, H),
                       (6, H), (7, 1000 - 7 * H)]:
            off = s * 1000 + i8 * H
            pltpu.sync_copy(s1h.at[pl.ds(off, sz)], exrow.at[pl.ds(0, sz)])
            pltpu.sync_copy(exrow.at[pl.ds(0, sz)], s1_sh.at[pl.ds(off, sz)])
            pltpu.sync_copy(s2h.at[pl.ds(off, sz)], exrow.at[pl.ds(0, sz)])
            pltpu.sync_copy(exrow.at[pl.ds(0, sz)], s2_sh.at[pl.ds(off, sz)])

    @pl.when(s == 10)
    def _stage_s2_tail():
        pltpu.sync_copy(denbuf.at[pl.ds(0, L)], s2_sh.at[pl.ds(N, L)])

    pltpu.sync_copy(dst2.at[pl.ds(ebase, RPT)], dstb)

    plsc.subcore_barrier()

    for c5 in range(RPT // L):
        pltpu.sync_copy(src2.at[pl.ds(ebase + c5 * L, L)], srcc.at[pl.ds(0, L)])

        def _erow(j16, carry):
            jj = c5 * L + j16
            pltpu.sync_copy(s1_sh.at[srcc.at[j16]], s1row)
            pltpu.sync_copy(s2_sh.at[dstb.at[jj]], s2row)
            for k in range(K8):
                sl = pl.ds(k * L, L)
                a = s1row[sl] + s2row[sl]
                e = jnp.maximum(a, 0.01 * a)
                exrow[sl] = jnp.exp(e)
            pltpu.sync_copy(exrow, den_sh.at[dstb.at[jj]], add=True)
            return carry

        lax.fori_loop(0, L, _erow, None)

    plsc.subcore_barrier()

    def _p4_step(j, carry):
        b = lax.rem(j, 2)
        bm = lax.rem(j + 1, 2)

        @pl.when(j < RPT)
        def _issue():
            j16 = lax.rem(j, L)
            cbase = pl.multiple_of(lax.rem(lax.div(j, L), 2) * L, L)
            crow = cbase + j16

            @pl.when(j16 == 0)
            def _stage_chunk():
                jm16 = pl.multiple_of(ebase + j, L)
                pltpu.sync_copy(src2.at[pl.ds(jm16, L)],
                                srcc.at[pl.ds(cbase, L)])

            pltpu.sync_copy(s1_sh.at[srcc.at[crow]], s1row)
            pltpu.sync_copy(s2_sh.at[dstb.at[j]], s2row)
            pltpu.sync_copy(den_sh.at[dstb.at[j]], dnrow)
            for k in range(K8):
                sl = pl.ds(k * L, L)
                srcc[crow, sl] = srcc[crow, sl] + coff
                a = s1row[sl] + s2row[sl]
                e = jnp.maximum(a, 0.01 * a)
                alrow[b, sl] = jnp.exp(e) / dnrow[sl]
            pltpu.async_copy(hs2.at[srcc.at[crow]], rowb.at[b], sem.at[b])

        @pl.when(j > 0)
        def _process():
            jm = j - 1
            pltpu.make_async_copy(hs2.at[pl.ds(0, H)], rowb.at[bm],
                                  sem.at[bm]).wait()

            def _scale(r, cr):
                av = plsc.load_gather(alrow, [_full16(bm), _full16(r)])
                for k in range(K8):
                    sl = pl.ds(k * L, L)
                    rowb[bm, r, sl] = rowb[bm, r, sl] * av
                return cr

            lax.fori_loop(0, H, _scale, None)
            pltpu.sync_copy(rowb.at[bm], acc_sh.at[dstb.at[jm]], add=True)

        return carry

    lax.fori_loop(0, RPT + 1, _p4_step, None)

    plsc.subcore_barrier()

    def _epi_chunk(rb, sz):
        pltpu.sync_copy(acc_sh.at[pl.ds(rb, sz)], rowb.at[0, pl.ds(0, sz)])
        pltpu.sync_copy(ht2.at[pl.ds(coff + rb, sz)], rowb.at[1, pl.ds(0, sz)])
        pltpu.sync_copy(den_sh.at[pl.ds(rb, sz)], denbuf.at[pl.ds(0, sz)])

        def _sel(r, cr):
            dn = plsc.load_gather(denbuf, [_full16(r)])
            m = dn > 0.0
            for k in range(K8):
                sl = pl.ds(k * L, L)
                rowb[0, r, sl] = jnp.where(m, rowb[0, r, sl], rowb[1, r, sl])
            return cr

        lax.fori_loop(0, sz, _sel, None)
        pltpu.sync_copy(rowb.at[0, pl.ds(0, sz)], out2.at[pl.ds(coff + rb, sz)])

    for k4 in range(4):
        _epi_chunk(s * RT + k4 * H, H)

    @pl.when(s < NS - 1)
    def _epi_tail():
        _epi_chunk(s * RT + 4 * H, RT - 4 * H)

    @pl.when(s == NS - 1)
    def _epi_last():
        _epi_chunk((NS - 1) * RT + 4 * H, N - ((NS - 1) * RT + 4 * H))


def kernel(h_sent, h_type, edge_index, attn_W):
    s1, s2 = _scores(h_sent, h_type, attn_W)

    zpad = jnp.zeros((NPH - N, H), jnp.float32)
    hs2 = jnp.concatenate([h_sent[:, :H], zpad, h_sent[:, H:], zpad], axis=0)
    ht2 = jnp.concatenate([h_type[:, :H], zpad, h_type[:, H:], zpad], axis=0)

    npad = EROWS * H - E
    pad_src = jnp.arange(npad, dtype=jnp.int32) % N
    pad_dst = N + (jnp.arange(npad, dtype=jnp.int32) % L)
    src2 = jnp.concatenate([edge_index[0], pad_src]).reshape(EROWS, H)
    dst2 = jnp.concatenate([edge_index[1], pad_dst]).reshape(EROWS, H)

    mesh = plsc.VectorSubcoreMesh(core_axis_name="c", subcore_axis_name="s",
                                  num_cores=2, num_subcores=NS)
    out2 = pl.kernel(
        _gat_sc_body,
        out_type=jax.ShapeDtypeStruct((2 * NPH, H), jnp.float32),
        mesh=mesh,
        compiler_params=pltpu.CompilerParams(needs_layout_passes=False),
        scratch_types=[
            pltpu.VMEM_SHARED((NPH, H), jnp.float32),
            pltpu.VMEM_SHARED((NPH,), jnp.float32),
            pltpu.VMEM_SHARED((N,), jnp.float32),
            pltpu.VMEM_SHARED((N + L,), jnp.float32),
            pltpu.VMEM((RPT, H), jnp.int32),
            pltpu.VMEM((2 * L, H), jnp.int32),
            pltpu.VMEM((2, H, H), jnp.float32),
            pltpu.VMEM((H,), jnp.float32),
            pltpu.VMEM((H,), jnp.float32),
            pltpu.VMEM((H,), jnp.float32),
            pltpu.VMEM((H,), jnp.float32),
            pltpu.VMEM((H,), jnp.float32),
            pltpu.VMEM((2, H), jnp.float32),
            pltpu.SemaphoreType.DMA((2,)),
        ],
    )(hs2, ht2, src2, dst2, s1, s2)

    return jnp.concatenate([out2[:N], out2[NPH:NPH + N]], axis=1)
